# baseline (speedup 1.0000x reference)
import jax
import jax.numpy as jnp
from jax import lax
from jax.experimental import pallas as pl
from jax.experimental.pallas import tpu as pltpu
from jax.experimental.pallas import tpu_sc as plsc

N = 10000
E = 160000
D = 128
NC = 2
NS = 16
L = 16

CHUNK = 128
E_PAD = 163840
EPT = E_PAD // NS
NCHUNK = EPT // CHUNK
NSLAB = 4
Q = NCHUNK // NSLAB
N_PAD = 10112
ZSEG = N_PAD // NS
WSTEP = 624
WSEG = 640
STG = 128
DL = 8
SPLIT = 4
PC = CHUNK // SPLIT


def _sc_kernel_body(x_user, x_item, src_f, dst_f, src_b, dst_b,
                    agg_f, deg_f, agg_b, deg_b,
                    src_v, dst_v, msg_v, ones_v, dstage_v,
                    acc_s, deg_s, gsem0, gsem1, ssem0, ssem1, dsem0, dsem1):
    c = lax.axis_index("c")
    s = lax.axis_index("s")
    gsem = (gsem0, gsem1)
    ssem = (ssem0, ssem1)
    dsem = (dsem0, dsem1)

    zero16 = jnp.zeros((L,), jnp.float32)
    oneD = jnp.ones((DL,), jnp.float32)
    zeroD = jnp.zeros((DL,), jnp.float32)

    @pl.loop(0, CHUNK)
    def fill_rows(i):
        for j in range(D // L):
            msg_v[0, i, pl.ds(j * L, L)] = zero16
        ones_v[i, :] = oneD
        dstage_v[i, :] = zeroD

    zb = s * ZSEG
    for rs in (0, STG, 2 * STG, 3 * STG, ZSEG - STG):
        pltpu.sync_copy(msg_v.at[0], acc_s.at[pl.ds(zb + rs, STG)])
        pltpu.sync_copy(dstage_v, deg_s.at[pl.ds(zb + rs, STG)])
    plsc.subcore_barrier()

    def do_relation(x_hbm, src_hbm, dst_hbm):
        def fire_gather(gi, b):
            for p in range(SPLIT):
                pltpu.async_copy(
                    x_hbm.at[src_v.at[gi, pl.ds(p * PC, PC)]],
                    msg_v.at[b, pl.ds(p * PC, PC)], gsem[b])

        def wait_gather(gi, b):
            for p in range(SPLIT):
                pltpu.make_async_copy(
                    x_hbm.at[src_v.at[gi, pl.ds(p * PC, PC)]],
                    msg_v.at[b, pl.ds(p * PC, PC)], gsem[b]).wait()

        for h in range(NSLAB):
            pltpu.sync_copy(src_hbm.at[s, pl.ds(h * Q, Q)], src_v)
            pltpu.sync_copy(dst_hbm.at[s, pl.ds(h * Q, Q)], dst_v)

            fire_gather(0, 0)

            @pl.loop(0, Q, step=2)
            def body(g):
                for b in range(2):
                    gi = g + b
                    ob = 1 - b
                    wait_gather(gi, b)

                    @pl.when(gi + 1 < Q)
                    def _():
                        @pl.when(gi >= 1)
                        def _():
                            pltpu.make_async_copy(
                                msg_v.at[ob], acc_s.at[dst_v.at[gi - 1]],
                                ssem[ob]).wait()
                            pltpu.make_async_copy(
                                ones_v, deg_s.at[dst_v.at[gi - 1]],
                                dsem[ob]).wait()
                        fire_gather(gi + 1, ob)

                    pltpu.async_copy(msg_v.at[b], acc_s.at[dst_v.at[gi]],
                                     ssem[b], add=True)
                    pltpu.async_copy(ones_v, deg_s.at[dst_v.at[gi]],
                                     dsem[b], add=True)

            for gi, b in ((Q - 2, 0), (Q - 1, 1)):
                pltpu.make_async_copy(
                    msg_v.at[b], acc_s.at[dst_v.at[gi]], ssem[b]).wait()
                pltpu.make_async_copy(
                    ones_v, deg_s.at[dst_v.at[gi]], dsem[b]).wait()

    @pl.when(c == 0)
    def _():
        do_relation(x_user, src_f, dst_f)

    @pl.when(c == 1)
    def _():
        do_relation(x_item, src_b, dst_b)

    plsc.subcore_barrier()

    def writeout(agg_hbm, deg_hbm):
        rb = s * WSTEP
        for rs in range(0, WSEG, STG):
            pltpu.sync_copy(acc_s.at[pl.ds(rb + rs, STG)], msg_v.at[0])
            pltpu.sync_copy(msg_v.at[0], agg_hbm.at[pl.ds(rb + rs, STG)])
            pltpu.sync_copy(deg_s.at[pl.ds(rb + rs, STG)], dstage_v)
            pltpu.sync_copy(dstage_v, deg_hbm.at[pl.ds(rb + rs, STG)])

    @pl.when(c == 0)
    def _():
        writeout(agg_f, deg_f)

    @pl.when(c == 1)
    def _():
        writeout(agg_b, deg_b)


def _make_sc_call():
    mesh = plsc.VectorSubcoreMesh(
        core_axis_name="c", subcore_axis_name="s",
        num_cores=NC, num_subcores=NS)
    out_type = (
        jax.ShapeDtypeStruct((N, D), jnp.float32),
        jax.ShapeDtypeStruct((N, DL), jnp.float32),
        jax.ShapeDtypeStruct((N, D), jnp.float32),
        jax.ShapeDtypeStruct((N, DL), jnp.float32),
    )
    scratch = [
        pltpu.VMEM((Q, CHUNK), jnp.int32),
        pltpu.VMEM((Q, CHUNK), jnp.int32),
        pltpu.VMEM((2, CHUNK, D), jnp.float32),
        pltpu.VMEM((CHUNK, DL), jnp.float32),
        pltpu.VMEM((STG, DL), jnp.float32),
        pltpu.VMEM_SHARED((N_PAD, D), jnp.float32),
        pltpu.VMEM_SHARED((N_PAD, DL), jnp.float32),
        pltpu.SemaphoreType.DMA,
        pltpu.SemaphoreType.DMA,
        pltpu.SemaphoreType.DMA,
        pltpu.SemaphoreType.DMA,
        pltpu.SemaphoreType.DMA,
        pltpu.SemaphoreType.DMA,
    ]
    return pl.kernel(_sc_kernel_body, out_type=out_type, mesh=mesh,
                     scratch_types=scratch,
                     compiler_params=pltpu.CompilerParams(
                         use_tc_tiling_on_sc=False))


def _tc_kernel_body(agg_f, deg_f, w_f, agg_b, deg_b, w_b, out_f, out_b):
    for agg, deg, w, out in ((agg_f, deg_f, w_f, out_f),
                             (agg_b, deg_b, w_b, out_b)):
        norm = 1.0 / jnp.maximum(deg[...][:, 0:1], 1.0)
        out[...] = jnp.dot(agg[...] * norm, w[...],
                           preferred_element_type=jnp.float32)


def _tc_call(agg_f, deg_f, w_f, agg_b, deg_b, w_b):
    rows = 1000
    grid = (N // rows,)
    mat_spec = pl.BlockSpec((rows, D), lambda i: (i, 0))
    deg_spec = pl.BlockSpec((rows, DL), lambda i: (i, 0))
    w_spec = pl.BlockSpec((D, D), lambda i: (0, 0))
    return pl.pallas_call(
        _tc_kernel_body,
        grid=grid,
        in_specs=[mat_spec, deg_spec, w_spec, mat_spec, deg_spec, w_spec],
        out_specs=[mat_spec, mat_spec],
        out_shape=[jax.ShapeDtypeStruct((N, D), jnp.float32),
                   jax.ShapeDtypeStruct((N, D), jnp.float32)],
    )(agg_f, deg_f, w_f, agg_b, deg_b, w_b)


def kernel(x_user, x_item, W_follows, W_bought,
           edge_index_follows, edge_index_bought):
    npad = E_PAD - E
    pad_src = jnp.zeros((npad,), jnp.int32)

    pad_dst = N + (jnp.arange(npad, dtype=jnp.int32) % (N_PAD - N))

    def pad_edges(edge_index):
        src = jnp.concatenate([edge_index[0], pad_src])
        dst = jnp.concatenate([edge_index[1], pad_dst])
        return src, dst

    def slab(a):
        return a.reshape(NS, NCHUNK, CHUNK)

    src_f, dst_f = pad_edges(edge_index_follows)
    src_b, dst_b = pad_edges(edge_index_bought)
    src_f, dst_f, src_b, dst_b = map(slab, (src_f, dst_f, src_b, dst_b))

    sc = _make_sc_call()
    agg_f, deg_f, agg_b, deg_b = sc(x_user, x_item, src_f, dst_f, src_b, dst_b)
    out_f, out_b = _tc_call(agg_f, deg_f, W_follows, agg_b, deg_b, W_bought)
    return (out_f, out_b)


# speedup vs baseline: 1.3940x; 1.3940x over previous
import jax
import jax.numpy as jnp
from jax import lax
from jax.experimental import pallas as pl
from jax.experimental.pallas import tpu as pltpu
from jax.experimental.pallas import tpu_sc as plsc

N = 10000
E = 160000
D = 128
NC = 2
NS = 16
L = 16

CHUNK = 128
E_PAD = 163840
EPT = E_PAD // NS
NCHUNK = EPT // CHUNK
NSLAB = 4
Q = NCHUNK // NSLAB
N_PAD = 10112
ZSEG = N_PAD // NS
WSTEP = 624
WSEG = 640
STG = 128
DL = 8
SPLIT = 4
PC = CHUNK // SPLIT


def _sc_kernel_body(x_user, x_item, src_f, dst_f, src_b, dst_b,
                    agg_f, deg_f, agg_b, deg_b,
                    src_v, dst_v, msg_v, ones_v, dstage_v,
                    acc_s, deg_s, gsem0, gsem1, ssem0, ssem1, dsem0, dsem1):
    c = lax.axis_index("c")
    s = lax.axis_index("s")
    gsem = (gsem0, gsem1)
    ssem = (ssem0, ssem1)
    dsem = (dsem0, dsem1)

    zero32 = jnp.zeros((2 * L,), jnp.bfloat16)
    oneD = jnp.ones((DL,), jnp.float32)
    zeroD = jnp.zeros((DL,), jnp.float32)

    @pl.loop(0, CHUNK)
    def fill_rows(i):
        for j in range(D // (2 * L)):
            msg_v[0, i, pl.ds(j * 2 * L, 2 * L)] = zero32
        ones_v[i, :] = oneD
        dstage_v[i, :] = zeroD

    zb = s * ZSEG
    for rs in (0, STG, 2 * STG, 3 * STG, ZSEG - STG):
        pltpu.sync_copy(msg_v.at[0], acc_s.at[pl.ds(zb + rs, STG)])
        pltpu.sync_copy(dstage_v, deg_s.at[pl.ds(zb + rs, STG)])
    plsc.subcore_barrier()

    def do_relation(x_hbm, src_hbm, dst_hbm):
        def fire_gather(gi, b):
            for p in range(SPLIT):
                pltpu.async_copy(
                    x_hbm.at[src_v.at[gi, pl.ds(p * PC, PC)]],
                    msg_v.at[b, pl.ds(p * PC, PC)], gsem[b])

        def wait_gather(gi, b):
            for p in range(SPLIT):
                pltpu.make_async_copy(
                    x_hbm.at[src_v.at[gi, pl.ds(p * PC, PC)]],
                    msg_v.at[b, pl.ds(p * PC, PC)], gsem[b]).wait()

        for h in range(NSLAB):
            pltpu.sync_copy(src_hbm.at[s, pl.ds(h * Q, Q)], src_v)
            pltpu.sync_copy(dst_hbm.at[s, pl.ds(h * Q, Q)], dst_v)

            fire_gather(0, 0)

            @pl.loop(0, Q, step=2)
            def body(g):
                for b in range(2):
                    gi = g + b
                    ob = 1 - b
                    wait_gather(gi, b)

                    @pl.when(gi + 1 < Q)
                    def _():
                        @pl.when(gi >= 1)
                        def _():
                            pltpu.make_async_copy(
                                msg_v.at[ob], acc_s.at[dst_v.at[gi - 1]],
                                ssem[ob]).wait()
                            pltpu.make_async_copy(
                                ones_v, deg_s.at[dst_v.at[gi - 1]],
                                dsem[ob]).wait()
                        fire_gather(gi + 1, ob)

                    pltpu.async_copy(msg_v.at[b], acc_s.at[dst_v.at[gi]],
                                     ssem[b], add=True)
                    pltpu.async_copy(ones_v, deg_s.at[dst_v.at[gi]],
                                     dsem[b], add=True)

            for gi, b in ((Q - 2, 0), (Q - 1, 1)):
                pltpu.make_async_copy(
                    msg_v.at[b], acc_s.at[dst_v.at[gi]], ssem[b]).wait()
                pltpu.make_async_copy(
                    ones_v, deg_s.at[dst_v.at[gi]], dsem[b]).wait()

    @pl.when(c == 0)
    def _():
        do_relation(x_user, src_f, dst_f)

    @pl.when(c == 1)
    def _():
        do_relation(x_item, src_b, dst_b)

    plsc.subcore_barrier()

    def writeout(agg_hbm, deg_hbm):
        rb = s * WSTEP
        for rs in range(0, WSEG, STG):
            pltpu.sync_copy(acc_s.at[pl.ds(rb + rs, STG)], msg_v.at[0])
            pltpu.sync_copy(msg_v.at[0], agg_hbm.at[pl.ds(rb + rs, STG)])
            pltpu.sync_copy(deg_s.at[pl.ds(rb + rs, STG)], dstage_v)
            pltpu.sync_copy(dstage_v, deg_hbm.at[pl.ds(rb + rs, STG)])

    @pl.when(c == 0)
    def _():
        writeout(agg_f, deg_f)

    @pl.when(c == 1)
    def _():
        writeout(agg_b, deg_b)


def _make_sc_call():
    mesh = plsc.VectorSubcoreMesh(
        core_axis_name="c", subcore_axis_name="s",
        num_cores=NC, num_subcores=NS)
    out_type = (
        jax.ShapeDtypeStruct((N, D), jnp.bfloat16),
        jax.ShapeDtypeStruct((N, DL), jnp.float32),
        jax.ShapeDtypeStruct((N, D), jnp.bfloat16),
        jax.ShapeDtypeStruct((N, DL), jnp.float32),
    )
    scratch = [
        pltpu.VMEM((Q, CHUNK), jnp.int32),
        pltpu.VMEM((Q, CHUNK), jnp.int32),
        pltpu.VMEM((2, CHUNK, D), jnp.bfloat16),
        pltpu.VMEM((CHUNK, DL), jnp.float32),
        pltpu.VMEM((STG, DL), jnp.float32),
        pltpu.VMEM_SHARED((N_PAD, D), jnp.bfloat16),
        pltpu.VMEM_SHARED((N_PAD, DL), jnp.float32),
        pltpu.SemaphoreType.DMA,
        pltpu.SemaphoreType.DMA,
        pltpu.SemaphoreType.DMA,
        pltpu.SemaphoreType.DMA,
        pltpu.SemaphoreType.DMA,
        pltpu.SemaphoreType.DMA,
    ]
    return pl.kernel(_sc_kernel_body, out_type=out_type, mesh=mesh,
                     scratch_types=scratch,
                     compiler_params=pltpu.CompilerParams(
                         use_tc_tiling_on_sc=False))


def _tc_kernel_body(agg_f, deg_f, w_f, agg_b, deg_b, w_b, out_f, out_b):
    for agg, deg, w, out in ((agg_f, deg_f, w_f, out_f),
                             (agg_b, deg_b, w_b, out_b)):
        norm = 1.0 / jnp.maximum(deg[...][:, 0:1], 1.0)
        a = agg[...].astype(jnp.float32)
        out[...] = jnp.dot(a * norm, w[...],
                           preferred_element_type=jnp.float32)


def _tc_call(agg_f, deg_f, w_f, agg_b, deg_b, w_b):
    rows = 1000
    grid = (N // rows,)
    mat_spec = pl.BlockSpec((rows, D), lambda i: (i, 0))
    deg_spec = pl.BlockSpec((rows, DL), lambda i: (i, 0))
    w_spec = pl.BlockSpec((D, D), lambda i: (0, 0))
    return pl.pallas_call(
        _tc_kernel_body,
        grid=grid,
        in_specs=[mat_spec, deg_spec, w_spec, mat_spec, deg_spec, w_spec],
        out_specs=[mat_spec, mat_spec],
        out_shape=[jax.ShapeDtypeStruct((N, D), jnp.float32),
                   jax.ShapeDtypeStruct((N, D), jnp.float32)],
    )(agg_f, deg_f, w_f, agg_b, deg_b, w_b)


def kernel(x_user, x_item, W_follows, W_bought,
           edge_index_follows, edge_index_bought):
    npad = E_PAD - E
    pad_src = jnp.zeros((npad,), jnp.int32)

    pad_dst = N + (jnp.arange(npad, dtype=jnp.int32) % (N_PAD - N))

    def pad_edges(edge_index):
        src = jnp.concatenate([edge_index[0], pad_src])
        dst = jnp.concatenate([edge_index[1], pad_dst])
        return src, dst

    def slab(a):
        return a.reshape(NS, NCHUNK, CHUNK)

    src_f, dst_f = pad_edges(edge_index_follows)
    src_b, dst_b = pad_edges(edge_index_bought)
    src_f, dst_f, src_b, dst_b = map(slab, (src_f, dst_f, src_b, dst_b))

    sc = _make_sc_call()
    agg_f, deg_f, agg_b, deg_b = sc(x_user.astype(jnp.bfloat16),
                                    x_item.astype(jnp.bfloat16),
                                    src_f, dst_f, src_b, dst_b)
    out_f, out_b = _tc_call(agg_f, deg_f, W_follows, agg_b, deg_b, W_bought)
    return (out_f, out_b)


# speedup vs baseline: 1.5770x; 1.1312x over previous
import jax
import jax.numpy as jnp
from jax import lax
from jax.experimental import pallas as pl
from jax.experimental.pallas import tpu as pltpu
from jax.experimental.pallas import tpu_sc as plsc

N = 10000
E = 160000
D = 128
NC = 2
NS = 16
L = 16

CHUNK = 128
E_PAD = 163840
EPT = E_PAD // NS
NCHUNK = EPT // CHUNK
NBUF = 4
N_PAD = 10112
ZSEG = N_PAD // NS
WSTEP = 624
WSEG = 640
STG = 128
DL = 8


def _sc_kernel_body(x_user, x_item, src_f, dst_f, src_b, dst_b,
                    agg_f, deg_f, agg_b, deg_b,
                    src_v, dst_v, msg_v, ones_v, dstage_v,
                    acc_s, deg_s, *sems):
    c = lax.axis_index("c")
    s = lax.axis_index("s")
    gsem = sems[0:NBUF]
    ssem = sems[NBUF:2 * NBUF]
    dsem = sems[2 * NBUF:3 * NBUF]

    zero32 = jnp.zeros((2 * L,), jnp.bfloat16)
    oneD = jnp.ones((DL,), jnp.float32)
    zeroD = jnp.zeros((DL,), jnp.float32)

    @pl.loop(0, CHUNK)
    def fill_rows(i):
        for j in range(D // (2 * L)):
            msg_v[0, i, pl.ds(j * 2 * L, 2 * L)] = zero32
        ones_v[i, :] = oneD
        dstage_v[i, :] = zeroD

    zb = s * ZSEG
    for rs in (0, STG, 2 * STG, 3 * STG, ZSEG - STG):
        pltpu.sync_copy(msg_v.at[0], acc_s.at[pl.ds(zb + rs, STG)])
        pltpu.sync_copy(dstage_v, deg_s.at[pl.ds(zb + rs, STG)])
    plsc.subcore_barrier()

    def do_relation(x_hbm, src_hbm, dst_hbm):
        def fire_gather(gi, b):
            pltpu.async_copy(x_hbm.at[src_v.at[gi]], msg_v.at[b], gsem[b])

        def wait_gather(gi, b):
            pltpu.make_async_copy(
                x_hbm.at[src_v.at[gi]], msg_v.at[b], gsem[b]).wait()

        pltpu.sync_copy(src_hbm.at[s], src_v)
        pltpu.sync_copy(dst_hbm.at[s], dst_v)

        for b in range(NBUF - 1):
            fire_gather(b, b)

        @pl.loop(0, NCHUNK, step=NBUF)
        def body(g):
            for b in range(NBUF):
                gi = g + b
                nb = (b + NBUF - 1) % NBUF
                wait_gather(gi, b)

                @pl.when(gi + NBUF - 1 < NCHUNK)
                def _():
                    @pl.when(gi >= 1)
                    def _():
                        pltpu.make_async_copy(
                            msg_v.at[nb], acc_s.at[dst_v.at[gi - 1]],
                            ssem[nb]).wait()
                        pltpu.make_async_copy(
                            ones_v, deg_s.at[dst_v.at[gi - 1]],
                            dsem[nb]).wait()
                    fire_gather(gi + NBUF - 1, nb)

                pltpu.async_copy(msg_v.at[b], acc_s.at[dst_v.at[gi]],
                                 ssem[b], add=True)
                pltpu.async_copy(ones_v, deg_s.at[dst_v.at[gi]],
                                 dsem[b], add=True)

        for gi in range(NCHUNK - NBUF, NCHUNK):
            b = gi % NBUF
            pltpu.make_async_copy(
                msg_v.at[b], acc_s.at[dst_v.at[gi]], ssem[b]).wait()
            pltpu.make_async_copy(
                ones_v, deg_s.at[dst_v.at[gi]], dsem[b]).wait()

    @pl.when(c == 0)
    def _():
        do_relation(x_user, src_f, dst_f)

    @pl.when(c == 1)
    def _():
        do_relation(x_item, src_b, dst_b)

    plsc.subcore_barrier()

    def writeout(agg_hbm, deg_hbm):
        rb = s * WSTEP
        for rs in range(0, WSEG, STG):
            pltpu.sync_copy(acc_s.at[pl.ds(rb + rs, STG)], msg_v.at[0])
            pltpu.sync_copy(msg_v.at[0], agg_hbm.at[pl.ds(rb + rs, STG)])
            pltpu.sync_copy(deg_s.at[pl.ds(rb + rs, STG)], dstage_v)
            pltpu.sync_copy(dstage_v, deg_hbm.at[pl.ds(rb + rs, STG)])

    @pl.when(c == 0)
    def _():
        writeout(agg_f, deg_f)

    @pl.when(c == 1)
    def _():
        writeout(agg_b, deg_b)


def _make_sc_call():
    mesh = plsc.VectorSubcoreMesh(
        core_axis_name="c", subcore_axis_name="s",
        num_cores=NC, num_subcores=NS)
    out_type = (
        jax.ShapeDtypeStruct((N, D), jnp.bfloat16),
        jax.ShapeDtypeStruct((N, DL), jnp.float32),
        jax.ShapeDtypeStruct((N, D), jnp.bfloat16),
        jax.ShapeDtypeStruct((N, DL), jnp.float32),
    )
    scratch = [
        pltpu.VMEM((NCHUNK, CHUNK), jnp.int32),
        pltpu.VMEM((NCHUNK, CHUNK), jnp.int32),
        pltpu.VMEM((NBUF, CHUNK, D), jnp.bfloat16),
        pltpu.VMEM((CHUNK, DL), jnp.float32),
        pltpu.VMEM((STG, DL), jnp.float32),
        pltpu.VMEM_SHARED((N_PAD, D), jnp.bfloat16),
        pltpu.VMEM_SHARED((N_PAD, DL), jnp.float32),
    ] + [pltpu.SemaphoreType.DMA] * (3 * NBUF)
    return pl.kernel(_sc_kernel_body, out_type=out_type, mesh=mesh,
                     scratch_types=scratch,
                     compiler_params=pltpu.CompilerParams(
                         use_tc_tiling_on_sc=False))


def _tc_kernel_body(agg_f, deg_f, w_f, agg_b, deg_b, w_b, out_f, out_b):
    for agg, deg, w, out in ((agg_f, deg_f, w_f, out_f),
                             (agg_b, deg_b, w_b, out_b)):
        norm = 1.0 / jnp.maximum(deg[...][:, 0:1], 1.0)
        a = agg[...].astype(jnp.float32)
        out[...] = jnp.dot(a * norm, w[...],
                           preferred_element_type=jnp.float32)


def _tc_call(agg_f, deg_f, w_f, agg_b, deg_b, w_b):
    rows = 1000
    grid = (N // rows,)
    mat_spec = pl.BlockSpec((rows, D), lambda i: (i, 0))
    deg_spec = pl.BlockSpec((rows, DL), lambda i: (i, 0))
    w_spec = pl.BlockSpec((D, D), lambda i: (0, 0))
    return pl.pallas_call(
        _tc_kernel_body,
        grid=grid,
        in_specs=[mat_spec, deg_spec, w_spec, mat_spec, deg_spec, w_spec],
        out_specs=[mat_spec, mat_spec],
        out_shape=[jax.ShapeDtypeStruct((N, D), jnp.float32),
                   jax.ShapeDtypeStruct((N, D), jnp.float32)],
    )(agg_f, deg_f, w_f, agg_b, deg_b, w_b)


def kernel(x_user, x_item, W_follows, W_bought,
           edge_index_follows, edge_index_bought):
    npad = E_PAD - E
    pad_src = jnp.zeros((npad,), jnp.int32)

    pad_dst = N + (jnp.arange(npad, dtype=jnp.int32) % (N_PAD - N))

    def pad_edges(edge_index):
        src = jnp.concatenate([edge_index[0], pad_src])
        dst = jnp.concatenate([edge_index[1], pad_dst])
        return src, dst

    def slab(a):
        return a.reshape(NS, NCHUNK, CHUNK)

    src_f, dst_f = pad_edges(edge_index_follows)
    src_b, dst_b = pad_edges(edge_index_bought)
    src_f, dst_f, src_b, dst_b = map(slab, (src_f, dst_f, src_b, dst_b))

    sc = _make_sc_call()
    agg_f, deg_f, agg_b, deg_b = sc(x_user.astype(jnp.bfloat16),
                                    x_item.astype(jnp.bfloat16),
                                    src_f, dst_f, src_b, dst_b)
    out_f, out_b = _tc_call(agg_f, deg_f, W_follows, agg_b, deg_b, W_bought)
    return (out_f, out_b)


# speedup vs baseline: 1.5776x; 1.0004x over previous
import jax
import jax.numpy as jnp
from jax import lax
from jax.experimental import pallas as pl
from jax.experimental.pallas import tpu as pltpu
from jax.experimental.pallas import tpu_sc as plsc

N = 10000
E = 160000
D = 128
NC = 2
NS = 16
L = 16

CHUNK = 128
E_PAD = 163840
EPT = E_PAD // NS
NCHUNK = EPT // CHUNK
NBUF = 5
N_PAD = 10112
ZSEG = N_PAD // NS
WSTEP = 624
WSEG = 640
STG = 128
DL = 8


def _sc_kernel_body(x_user, x_item, src_f, dst_f, src_b, dst_b,
                    agg_f, deg_f, agg_b, deg_b,
                    src_v, dst_v, msg_v, ones_v, dstage_v,
                    acc_s, deg_s, *sems):
    c = lax.axis_index("c")
    s = lax.axis_index("s")
    gsem = sems[0:NBUF]
    ssem = sems[NBUF:2 * NBUF]
    dsem = sems[2 * NBUF:3 * NBUF]

    zero32 = jnp.zeros((2 * L,), jnp.bfloat16)
    oneD = jnp.ones((DL,), jnp.float32)
    zeroD = jnp.zeros((DL,), jnp.float32)

    @pl.loop(0, CHUNK)
    def fill_rows(i):
        for j in range(D // (2 * L)):
            msg_v[0, i, pl.ds(j * 2 * L, 2 * L)] = zero32
        ones_v[i, :] = oneD
        dstage_v[i, :] = zeroD

    zb = s * ZSEG
    for rs in (0, STG, 2 * STG, 3 * STG, ZSEG - STG):
        pltpu.sync_copy(msg_v.at[0], acc_s.at[pl.ds(zb + rs, STG)])
        pltpu.sync_copy(dstage_v, deg_s.at[pl.ds(zb + rs, STG)])
    plsc.subcore_barrier()

    def do_relation(x_hbm, src_hbm, dst_hbm):
        def fire_gather(gi, b):
            pltpu.async_copy(x_hbm.at[src_v.at[gi]], msg_v.at[b], gsem[b])

        def wait_gather(gi, b):
            pltpu.make_async_copy(
                x_hbm.at[src_v.at[gi]], msg_v.at[b], gsem[b]).wait()

        pltpu.sync_copy(src_hbm.at[s], src_v)
        pltpu.sync_copy(dst_hbm.at[s], dst_v)

        for b in range(NBUF - 1):
            fire_gather(b, b)

        @pl.loop(0, NCHUNK, step=NBUF)
        def body(g):
            for b in range(NBUF):
                gi = g + b
                nb = (b + NBUF - 1) % NBUF
                wait_gather(gi, b)

                @pl.when(gi + NBUF - 1 < NCHUNK)
                def _():
                    @pl.when(gi >= 1)
                    def _():
                        pltpu.make_async_copy(
                            msg_v.at[nb], acc_s.at[dst_v.at[gi - 1]],
                            ssem[nb]).wait()
                        pltpu.make_async_copy(
                            ones_v, deg_s.at[dst_v.at[gi - 1]],
                            dsem[nb]).wait()
                    fire_gather(gi + NBUF - 1, nb)

                pltpu.async_copy(msg_v.at[b], acc_s.at[dst_v.at[gi]],
                                 ssem[b], add=True)
                pltpu.async_copy(ones_v, deg_s.at[dst_v.at[gi]],
                                 dsem[b], add=True)

        for gi in range(NCHUNK - NBUF, NCHUNK):
            b = gi % NBUF
            pltpu.make_async_copy(
                msg_v.at[b], acc_s.at[dst_v.at[gi]], ssem[b]).wait()
            pltpu.make_async_copy(
                ones_v, deg_s.at[dst_v.at[gi]], dsem[b]).wait()

    @pl.when(c == 0)
    def _():
        do_relation(x_user, src_f, dst_f)

    @pl.when(c == 1)
    def _():
        do_relation(x_item, src_b, dst_b)

    plsc.subcore_barrier()

    def writeout(agg_hbm, deg_hbm):
        rb = s * WSTEP
        for rs in range(0, WSEG, STG):
            pltpu.sync_copy(acc_s.at[pl.ds(rb + rs, STG)], msg_v.at[0])
            pltpu.sync_copy(msg_v.at[0], agg_hbm.at[pl.ds(rb + rs, STG)])
            pltpu.sync_copy(deg_s.at[pl.ds(rb + rs, STG)], dstage_v)
            pltpu.sync_copy(dstage_v, deg_hbm.at[pl.ds(rb + rs, STG)])

    @pl.when(c == 0)
    def _():
        writeout(agg_f, deg_f)

    @pl.when(c == 1)
    def _():
        writeout(agg_b, deg_b)


def _make_sc_call():
    mesh = plsc.VectorSubcoreMesh(
        core_axis_name="c", subcore_axis_name="s",
        num_cores=NC, num_subcores=NS)
    out_type = (
        jax.ShapeDtypeStruct((N, D), jnp.bfloat16),
        jax.ShapeDtypeStruct((N, DL), jnp.float32),
        jax.ShapeDtypeStruct((N, D), jnp.bfloat16),
        jax.ShapeDtypeStruct((N, DL), jnp.float32),
    )
    scratch = [
        pltpu.VMEM((NCHUNK, CHUNK), jnp.int32),
        pltpu.VMEM((NCHUNK, CHUNK), jnp.int32),
        pltpu.VMEM((NBUF, CHUNK, D), jnp.bfloat16),
        pltpu.VMEM((CHUNK, DL), jnp.float32),
        pltpu.VMEM((STG, DL), jnp.float32),
        pltpu.VMEM_SHARED((N_PAD, D), jnp.bfloat16),
        pltpu.VMEM_SHARED((N_PAD, DL), jnp.float32),
    ] + [pltpu.SemaphoreType.DMA] * (3 * NBUF)
    return pl.kernel(_sc_kernel_body, out_type=out_type, mesh=mesh,
                     scratch_types=scratch,
                     compiler_params=pltpu.CompilerParams(
                         use_tc_tiling_on_sc=False))


def _tc_kernel_body(agg_f, deg_f, w_f, agg_b, deg_b, w_b, out_f, out_b):
    for agg, deg, w, out in ((agg_f, deg_f, w_f, out_f),
                             (agg_b, deg_b, w_b, out_b)):
        norm = 1.0 / jnp.maximum(deg[...][:, 0:1], 1.0)
        a = agg[...].astype(jnp.float32)
        out[...] = jnp.dot(a * norm, w[...],
                           preferred_element_type=jnp.float32)


def _tc_call(agg_f, deg_f, w_f, agg_b, deg_b, w_b):
    rows = 1000
    grid = (N // rows,)
    mat_spec = pl.BlockSpec((rows, D), lambda i: (i, 0))
    deg_spec = pl.BlockSpec((rows, DL), lambda i: (i, 0))
    w_spec = pl.BlockSpec((D, D), lambda i: (0, 0))
    return pl.pallas_call(
        _tc_kernel_body,
        grid=grid,
        in_specs=[mat_spec, deg_spec, w_spec, mat_spec, deg_spec, w_spec],
        out_specs=[mat_spec, mat_spec],
        out_shape=[jax.ShapeDtypeStruct((N, D), jnp.float32),
                   jax.ShapeDtypeStruct((N, D), jnp.float32)],
    )(agg_f, deg_f, w_f, agg_b, deg_b, w_b)


def kernel(x_user, x_item, W_follows, W_bought,
           edge_index_follows, edge_index_bought):
    npad = E_PAD - E
    pad_src = jnp.zeros((npad,), jnp.int32)

    pad_dst = N + (jnp.arange(npad, dtype=jnp.int32) % (N_PAD - N))

    def pad_edges(edge_index):
        src = jnp.concatenate([edge_index[0], pad_src])
        dst = jnp.concatenate([edge_index[1], pad_dst])
        return src, dst

    def slab(a):
        return a.reshape(NS, NCHUNK, CHUNK)

    src_f, dst_f = pad_edges(edge_index_follows)
    src_b, dst_b = pad_edges(edge_index_bought)
    src_f, dst_f, src_b, dst_b = map(slab, (src_f, dst_f, src_b, dst_b))

    sc = _make_sc_call()
    agg_f, deg_f, agg_b, deg_b = sc(x_user.astype(jnp.bfloat16),
                                    x_item.astype(jnp.bfloat16),
                                    src_f, dst_f, src_b, dst_b)
    out_f, out_b = _tc_call(agg_f, deg_f, W_follows, agg_b, deg_b, W_bought)
    return (out_f, out_b)
